# Initial kernel scaffold; baseline (speedup 1.0000x reference)
import jax
import jax.numpy as jnp
from jax.experimental import pallas as pl


def kernel(inputs):
    raise NotImplementedError("write your pallas kernel here")


# speedup vs baseline: 1.2698x; 1.2698x over previous
import functools

import jax
import jax.numpy as jnp
from jax import lax
from jax.experimental import pallas as pl
from jax.experimental.pallas import tpu as pltpu
from jax.experimental.pallas import tpu_sc as plsc

_SEGMENTS = [-2.0, -1.5, -1.0, -0.5, 0.0, 0.5, 1.0, 1.5, 2.0]
_LAST = float(_SEGMENTS[-1])
_LO = len(_SEGMENTS)
_HI = len(_SEGMENTS) + 1

_N = 4194304
_NC = 2
_NS = 16
_LANES = 16
_NW = _NC * _NS
_PER_W = _N // _NW
_CHUNK = 16384
_NCHUNK = _PER_W // _CHUNK


def _sc_body(x_hbm, out_hbm, in_v, out_v):
    wid = lax.axis_index("s") * _NC + lax.axis_index("c")
    base = wid * _PER_W

    def chunk_body(cidx, carry):
        off = base + cidx * _CHUNK
        pltpu.sync_copy(x_hbm.at[pl.ds(off, _CHUNK)], in_v)

        def vec_body(i, c):
            x = in_v[pl.ds(i * _LANES, _LANES)]
            lo = jnp.full((_LANES,), _LO, jnp.int32)
            hi = jnp.full((_LANES,), _HI, jnp.int32)
            out_v[pl.ds(i * _LANES, _LANES)] = jnp.where(x < _LAST, lo, hi)
            return c

        lax.fori_loop(0, _CHUNK // _LANES, vec_body, 0)
        pltpu.sync_copy(out_v, out_hbm.at[pl.ds(off, _CHUNK)])
        return carry

    lax.fori_loop(0, _NCHUNK, chunk_body, 0)


_sc_call = functools.partial(
    pl.kernel,
    mesh=plsc.VectorSubcoreMesh(core_axis_name="c", subcore_axis_name="s"),
    out_type=jax.ShapeDtypeStruct((_N,), jnp.int32),
    scratch_types=[
        pltpu.VMEM((_CHUNK,), jnp.float32),
        pltpu.VMEM((_CHUNK,), jnp.int32),
    ],
)(_sc_body)


def kernel(inputs):
    flat = inputs.reshape(_N)
    out = _sc_call(flat)
    return out.reshape(_N, 1)


# speedup vs baseline: 2.4780x; 1.9514x over previous
import functools

import jax
import jax.numpy as jnp
from jax import lax
from jax.experimental import pallas as pl
from jax.experimental.pallas import tpu as pltpu
from jax.experimental.pallas import tpu_sc as plsc

_SEGMENTS = [-2.0, -1.5, -1.0, -0.5, 0.0, 0.5, 1.0, 1.5, 2.0]
_LAST = float(_SEGMENTS[-1])
_LO = len(_SEGMENTS)
_HI = len(_SEGMENTS) + 1

_N = 4194304
_NC = 2
_NS = 16
_LANES = 16
_NW = _NC * _NS
_PER_W = _N // _NW
_CHUNK = 16384
_NCHUNK = _PER_W // _CHUNK


def _sc_body(x_hbm, out_hbm, in_v0, in_v1, out_v0, out_v1,
             sem_i0, sem_i1, sem_o0, sem_o1):
    wid = lax.axis_index("s") * _NC + lax.axis_index("c")
    base = wid * _PER_W

    in_bufs = (in_v0, in_v1)
    out_bufs = (out_v0, out_v1)
    in_sems = (sem_i0, sem_i1)
    out_sems = (sem_o0, sem_o1)

    lo = jnp.full((_LANES,), _LO, jnp.int32)
    hi = jnp.full((_LANES,), _HI, jnp.int32)

    def start_in(c):
        return pltpu.async_copy(
            x_hbm.at[pl.ds(base + c * _CHUNK, _CHUNK)],
            in_bufs[c % 2], in_sems[c % 2])

    def start_out(c):
        return pltpu.async_copy(
            out_bufs[c % 2],
            out_hbm.at[pl.ds(base + c * _CHUNK, _CHUNK)],
            out_sems[c % 2])

    h_in = [start_in(0), start_in(1)]
    h_out = [None, None]
    for c in range(_NCHUNK):
        b = c % 2
        if h_out[b] is not None:
            h_out[b].wait()
        h_in[b].wait()
        in_b, out_b = in_bufs[b], out_bufs[b]

        @plsc.parallel_loop(0, _CHUNK, step=_LANES, unroll=8)
        def _(i):
            x = in_b[pl.ds(i, _LANES)]
            out_b[pl.ds(i, _LANES)] = jnp.where(x < _LAST, lo, hi)

        h_out[b] = start_out(c)
        if c + 2 < _NCHUNK:
            h_in[b] = start_in(c + 2)
    h_out[0].wait()
    h_out[1].wait()


_sc_call = functools.partial(
    pl.kernel,
    mesh=plsc.VectorSubcoreMesh(core_axis_name="c", subcore_axis_name="s"),
    out_type=jax.ShapeDtypeStruct((_N,), jnp.int32),
    scratch_types=[
        pltpu.VMEM((_CHUNK,), jnp.float32),
        pltpu.VMEM((_CHUNK,), jnp.float32),
        pltpu.VMEM((_CHUNK,), jnp.int32),
        pltpu.VMEM((_CHUNK,), jnp.int32),
        pltpu.SemaphoreType.DMA,
        pltpu.SemaphoreType.DMA,
        pltpu.SemaphoreType.DMA,
        pltpu.SemaphoreType.DMA,
    ],
)(_sc_body)


def kernel(inputs):
    flat = inputs.reshape(_N)
    out = _sc_call(flat)
    return out.reshape(_N, 1)
